# Initial kernel scaffold; baseline (speedup 1.0000x reference)
import jax
import jax.numpy as jnp
from jax.experimental import pallas as pl


def kernel(x, pos, edge_index, lW1, lb1, lW2, lb2, gW1, gb1, gW2, gb2, gW3, gb3, cW1, cb1, cW2, cb2):
    raise NotImplementedError("write your pallas kernel here")


# speedup vs baseline: 1.7066x; 1.7066x over previous
import functools

import jax
import jax.numpy as jnp
from jax.experimental import pallas as pl

N = 10000
E = 320000
D = 128
H = 256
OUT = 128

EBLK = 2560


def _edge_mlp_body(a_ref, w_ref, b_ref, o_ref):
    a = jnp.maximum(a_ref[...], 0.0)
    o_ref[...] = jnp.dot(a, w_ref[...], preferred_element_type=jnp.float32) + b_ref[...]


def _edge_mlp(a, w, b):
    e = a.shape[0]
    grid = e // EBLK
    return pl.pallas_call(
        _edge_mlp_body,
        grid=(grid,),
        in_specs=[
            pl.BlockSpec((EBLK, H), lambda i: (i, 0)),
            pl.BlockSpec((H, H), lambda i: (0, 0)),
            pl.BlockSpec((1, H), lambda i: (0, 0)),
        ],
        out_specs=pl.BlockSpec((EBLK, H), lambda i: (i, 0)),
        out_shape=jax.ShapeDtypeStruct((e, H), jnp.float32),
    )(a, w, b.reshape(1, H))


def kernel(x, pos, edge_index, lW1, lb1, lW2, lb2, gW1, gb1, gW2, gb2, gW3, gb3, cW1, cb1, cW2, cb2):
    n = x.shape[0]
    src = edge_index[0]
    dst = edge_index[1]

    xw = x @ lW1[:D] + lb1
    pw = pos @ lW1[D:]
    u = xw + pw

    a_e = u[src] - pw[dst]
    h_e = _edge_mlp(a_e, lW2, lb2)
    self_h = jnp.maximum(xw, 0.0) @ lW2 + lb2

    agg = jax.ops.segment_max(h_e, dst, num_segments=n)
    agg = jnp.maximum(jnp.where(jnp.isfinite(agg), agg, -jnp.inf), self_h)

    g = jnp.maximum(agg @ gW1 + gb1, 0.0)
    g = jnp.maximum(g @ gW2 + gb2, 0.0)
    h0 = g @ gW3 + gb3

    deg = jax.ops.segment_sum(jnp.ones((E,), jnp.float32), dst, num_segments=n) + 1.0
    dis = jax.lax.rsqrt(deg)[:, None]

    xw1 = h0 @ cW1
    z1 = dis * xw1
    acc1 = jax.ops.segment_sum(z1[src], dst, num_segments=n)
    h1 = jnp.maximum(dis * acc1 + dis * dis * xw1 + cb1, 0.0)

    xw2 = h1 @ cW2
    z2 = dis * xw2
    acc2 = jax.ops.segment_sum(z2[src], dst, num_segments=n)
    h2 = jnp.maximum(dis * acc2 + dis * dis * xw2 + cb2, 0.0)
    return h2


# speedup vs baseline: 2.3193x; 1.3590x over previous
import functools

import jax
import jax.numpy as jnp
from jax import lax
from jax.experimental import pallas as pl
from jax.experimental.pallas import tpu as pltpu
from jax.experimental.pallas import tpu_sc as plsc

N = 10000
E = 320000
D = 128
H = 256
OUT = 128

EBLK = 2560

NC = 2
NS = 16
NW = NC * NS
L = 16

_SC_MESH = dict(core_axis_name="c", subcore_axis_name="s")


def _wid():
    return lax.axis_index("s") * NC + lax.axis_index("c")


EPW = E // NW
GCBS = 200


def _edge_gather_body(u_hbm, pw_hbm, src_hbm, dst_hbm, r_hbm,
                      sidx, didx, ubuf, pbuf, sem1, sem2):
    base = _wid() * EPW

    def chunk(i, carry):
        off = base + i * GCBS
        pltpu.sync_copy(src_hbm.at[pl.ds(off, GCBS)], sidx)
        pltpu.sync_copy(dst_hbm.at[pl.ds(off, GCBS)], didx)
        cu = pltpu.async_copy(u_hbm.at[sidx], ubuf, sem1)
        cp = pltpu.async_copy(pw_hbm.at[didx], pbuf, sem2)
        cu.wait()
        cp.wait()

        def row(r, c2):
            for c in range(H // L):
                s = pl.ds(c * L, L)
                ubuf[r, s] = jnp.maximum(ubuf[r, s] - pbuf[r, s], 0.0)
            return c2

        lax.fori_loop(0, GCBS, row, 0)
        pltpu.sync_copy(ubuf, r_hbm.at[pl.ds(off, GCBS)])
        return carry

    lax.fori_loop(0, EPW // GCBS, chunk, 0)


def _edge_gather(u, pw, src, dst):
    return pl.kernel(
        _edge_gather_body,
        out_type=jax.ShapeDtypeStruct((E, H), jnp.float32),
        mesh=plsc.VectorSubcoreMesh(**_SC_MESH),
        scratch_types=[
            pltpu.VMEM((GCBS,), jnp.int32),
            pltpu.VMEM((GCBS,), jnp.int32),
            pltpu.VMEM((GCBS, H), jnp.float32),
            pltpu.VMEM((GCBS, H), jnp.float32),
            pltpu.SemaphoreType.DMA,
            pltpu.SemaphoreType.DMA,
        ],
    )(u, pw, src, dst)


def _edge_mlp_body(a_ref, w_ref, b_ref, o_ref):
    a = jnp.maximum(a_ref[...], 0.0)
    o_ref[...] = jnp.dot(a, w_ref[...], preferred_element_type=jnp.float32) + b_ref[...]


def _edge_mlp(a, w, b):
    e = a.shape[0]
    grid = e // EBLK
    return pl.pallas_call(
        _edge_mlp_body,
        grid=(grid,),
        in_specs=[
            pl.BlockSpec((EBLK, H), lambda i: (i, 0)),
            pl.BlockSpec((H, H), lambda i: (0, 0)),
            pl.BlockSpec((1, H), lambda i: (0, 0)),
        ],
        out_specs=pl.BlockSpec((EBLK, H), lambda i: (i, 0)),
        out_shape=jax.ShapeDtypeStruct((e, H), jnp.float32),
    )(a, w, b.reshape(1, H))


def kernel(x, pos, edge_index, lW1, lb1, lW2, lb2, gW1, gb1, gW2, gb2, gW3, gb3, cW1, cb1, cW2, cb2):
    n = x.shape[0]
    src = edge_index[0]
    dst = edge_index[1]

    xw = x @ lW1[:D] + lb1
    pw = pos @ lW1[D:]
    u = xw + pw

    r_e = _edge_gather(u, pw, src, dst)
    h_e = _edge_mlp(r_e, lW2, lb2)
    self_h = jnp.maximum(xw, 0.0) @ lW2 + lb2

    agg = jax.ops.segment_max(h_e, dst, num_segments=n)
    agg = jnp.maximum(jnp.where(jnp.isfinite(agg), agg, -jnp.inf), self_h)

    g = jnp.maximum(agg @ gW1 + gb1, 0.0)
    g = jnp.maximum(g @ gW2 + gb2, 0.0)
    h0 = g @ gW3 + gb3

    deg = jax.ops.segment_sum(jnp.ones((E,), jnp.float32), dst, num_segments=n) + 1.0
    dis = jax.lax.rsqrt(deg)[:, None]

    xw1 = h0 @ cW1
    z1 = dis * xw1
    acc1 = jax.ops.segment_sum(z1[src], dst, num_segments=n)
    h1 = jnp.maximum(dis * acc1 + dis * dis * xw1 + cb1, 0.0)

    xw2 = h1 @ cW2
    z2 = dis * xw2
    acc2 = jax.ops.segment_sum(z2[src], dst, num_segments=n)
    h2 = jnp.maximum(dis * acc2 + dis * dis * xw2 + cb2, 0.0)
    return h2


# speedup vs baseline: 4.5110x; 1.9450x over previous
import functools

import jax
import jax.numpy as jnp
from jax import lax
from jax.experimental import pallas as pl
from jax.experimental.pallas import tpu as pltpu
from jax.experimental.pallas import tpu_sc as plsc

N = 10000
E = 320000
D = 128
H = 256
OUT = 128

EBLK = 2560

NC = 2
NS = 16
NW = NC * NS
L = 16

_SC_MESH = dict(core_axis_name="c", subcore_axis_name="s")


def _wid():
    return lax.axis_index("s") * NC + lax.axis_index("c")


EPW = E // NW
GCBS = 200


def _edge_gather_body(u_hbm, pw_hbm, src_hbm, dst_hbm, r_hbm,
                      sidx, didx, ubuf, pbuf, sem1, sem2):
    base = _wid() * EPW

    def chunk(i, carry):
        off = base + i * GCBS
        pltpu.sync_copy(src_hbm.at[pl.ds(off, GCBS)], sidx)
        pltpu.sync_copy(dst_hbm.at[pl.ds(off, GCBS)], didx)
        cu = pltpu.async_copy(u_hbm.at[sidx], ubuf, sem1)
        cp = pltpu.async_copy(pw_hbm.at[didx], pbuf, sem2)
        cu.wait()
        cp.wait()

        def row(r, c2):
            for c in range(H // L):
                s = pl.ds(c * L, L)
                ubuf[r, s] = jnp.maximum(ubuf[r, s] - pbuf[r, s], 0.0)
            return c2

        lax.fori_loop(0, GCBS, row, 0)
        pltpu.sync_copy(ubuf, r_hbm.at[pl.ds(off, GCBS)])
        return carry

    lax.fori_loop(0, EPW // GCBS, chunk, 0)


def _edge_gather(u, pw, src, dst):
    return pl.kernel(
        _edge_gather_body,
        out_type=jax.ShapeDtypeStruct((E, H), jnp.float32),
        mesh=plsc.VectorSubcoreMesh(**_SC_MESH),
        scratch_types=[
            pltpu.VMEM((GCBS,), jnp.int32),
            pltpu.VMEM((GCBS,), jnp.int32),
            pltpu.VMEM((GCBS, H), jnp.float32),
            pltpu.VMEM((GCBS, H), jnp.float32),
            pltpu.SemaphoreType.DMA,
            pltpu.SemaphoreType.DMA,
        ],
    )(u, pw, src, dst)


SCBS = 200


def _segsum_run(z_hbm, out_hbm, src_hbm, dst_hbm, acc_sh, sidx, didx, buf, sem):
    sid = lax.axis_index("s")
    ept = E // NS

    @pl.when(sid < 10)
    def _():
        pltpu.sync_copy(z_hbm.at[pl.ds(sid * 1000, 1000)],
                        acc_sh.at[pl.ds(sid * 1000, 1000)])

    plsc.subcore_barrier()

    def chunk(i, carry):
        off = sid * ept + i * SCBS
        pltpu.sync_copy(src_hbm.at[pl.ds(off, SCBS)], sidx)
        pltpu.sync_copy(dst_hbm.at[pl.ds(off, SCBS)], didx)
        pltpu.async_copy(z_hbm.at[sidx], buf, sem).wait()
        pltpu.async_copy(buf, acc_sh.at[didx], sem, add=True).wait()
        return carry

    lax.fori_loop(0, ept // SCBS, chunk, 0)
    plsc.subcore_barrier()

    @pl.when(sid < 10)
    def _():
        pltpu.sync_copy(acc_sh.at[pl.ds(sid * 1000, 1000)],
                        out_hbm.at[pl.ds(sid * 1000, 1000)])


def _segsum_body(zl_hbm, zr_hbm, src_hbm, dst_hbm, outl_hbm, outr_hbm,
                 acc_sh, sidx, didx, buf, sem):
    c = lax.axis_index("c")

    @pl.when(c == 0)
    def _():
        _segsum_run(zl_hbm, outl_hbm, src_hbm, dst_hbm, acc_sh, sidx, didx, buf, sem)

    @pl.when(c == 1)
    def _():
        _segsum_run(zr_hbm, outr_hbm, src_hbm, dst_hbm, acc_sh, sidx, didx, buf, sem)


def _segsum(zl, zr, src, dst, hc2):
    return pl.kernel(
        _segsum_body,
        out_type=(jax.ShapeDtypeStruct((N, hc2), jnp.float32),
                  jax.ShapeDtypeStruct((N, hc2), jnp.float32)),
        mesh=plsc.VectorSubcoreMesh(**_SC_MESH),
        scratch_types=[
            pltpu.VMEM_SHARED((N, hc2), jnp.float32),
            pltpu.VMEM((SCBS,), jnp.int32),
            pltpu.VMEM((SCBS,), jnp.int32),
            pltpu.VMEM((SCBS, hc2), jnp.float32),
            pltpu.SemaphoreType.DMA,
        ],
    )(zl, zr, src, dst)


def _segsum_es_body(z_hbm, src_hbm, dst_hbm, out_hbm, acc_sh, sidx, didx, buf, sem):
    c = lax.axis_index("c")
    sid = lax.axis_index("s")

    @pl.when(sid < 10)
    def _():
        pltpu.sync_copy(z_hbm.at[pl.ds(sid * 1000, 1000)],
                        acc_sh.at[pl.ds(sid * 1000, 1000)])

    plsc.subcore_barrier()

    def chunk(i, carry):
        off = (c * NS + sid) * EPW + i * SCBS
        pltpu.sync_copy(src_hbm.at[pl.ds(off, SCBS)], sidx)
        pltpu.sync_copy(dst_hbm.at[pl.ds(off, SCBS)], didx)
        pltpu.async_copy(z_hbm.at[sidx], buf, sem).wait()
        pltpu.async_copy(buf, acc_sh.at[didx], sem, add=True).wait()
        return carry

    lax.fori_loop(0, EPW // SCBS, chunk, 0)
    plsc.subcore_barrier()

    @pl.when(sid < 10)
    def _():
        pltpu.sync_copy(acc_sh.at[pl.ds(sid * 1000, 1000)],
                        out_hbm.at[c, pl.ds(sid * 1000, 1000)])


def _segsum_es(z, src, dst, w):
    parts = pl.kernel(
        _segsum_es_body,
        out_type=jax.ShapeDtypeStruct((NC, N, w), jnp.float32),
        mesh=plsc.VectorSubcoreMesh(**_SC_MESH),
        scratch_types=[
            pltpu.VMEM_SHARED((N, w), jnp.float32),
            pltpu.VMEM((SCBS,), jnp.int32),
            pltpu.VMEM((SCBS,), jnp.int32),
            pltpu.VMEM((SCBS, w), jnp.float32),
            pltpu.SemaphoreType.DMA,
        ],
    )(z, src, dst)
    return parts[0] + parts[1] - z


DCBS = 2000


def _deg_body(dst_hbm, out_hbm, cnt, dchunk):
    w = _wid()

    def z16(i, carry):
        cnt[pl.ds(i * L, L)] = jnp.zeros((L,), jnp.float32)
        return carry

    lax.fori_loop(0, N // L + 1, z16, 0)
    ones = jnp.ones((L,), jnp.float32)

    def chunk(i, carry):
        off = w * EPW + i * DCBS
        pltpu.sync_copy(dst_hbm.at[pl.ds(off, DCBS)], dchunk)

        def grp(j, cc):
            idx = dchunk[pl.ds(j * L, L)]
            plsc.addupdate_scatter(cnt, [idx], ones)
            return cc

        lax.fori_loop(0, DCBS // L, grp, 0)
        return carry

    lax.fori_loop(0, EPW // DCBS, chunk, 0)
    pltpu.sync_copy(cnt, out_hbm.at[w])


def _deg_counts(dst):
    return pl.kernel(
        _deg_body,
        out_type=jax.ShapeDtypeStruct((NW, N + L), jnp.float32),
        mesh=plsc.VectorSubcoreMesh(**_SC_MESH),
        scratch_types=[
            pltpu.VMEM((N + L,), jnp.float32),
            pltpu.VMEM((DCBS,), jnp.int32),
        ],
    )(dst)


def _edge_mlp_body(a_ref, w_ref, b_ref, o_ref):
    a = jnp.maximum(a_ref[...], 0.0)
    o_ref[...] = jnp.dot(a, w_ref[...], preferred_element_type=jnp.float32) + b_ref[...]


def _edge_mlp(a, w, b):
    e = a.shape[0]
    grid = e // EBLK
    return pl.pallas_call(
        _edge_mlp_body,
        grid=(grid,),
        in_specs=[
            pl.BlockSpec((EBLK, H), lambda i: (i, 0)),
            pl.BlockSpec((H, H), lambda i: (0, 0)),
            pl.BlockSpec((1, H), lambda i: (0, 0)),
        ],
        out_specs=pl.BlockSpec((EBLK, H), lambda i: (i, 0)),
        out_shape=jax.ShapeDtypeStruct((e, H), jnp.float32),
    )(a, w, b.reshape(1, H))


def kernel(x, pos, edge_index, lW1, lb1, lW2, lb2, gW1, gb1, gW2, gb2, gW3, gb3, cW1, cb1, cW2, cb2):
    n = x.shape[0]
    src = edge_index[0]
    dst = edge_index[1]

    xw = x @ lW1[:D] + lb1
    pw = pos @ lW1[D:]
    u = xw + pw

    r_e = _edge_gather(u, pw, src, dst)
    h_e = _edge_mlp(r_e, lW2, lb2)
    self_h = jnp.maximum(xw, 0.0) @ lW2 + lb2

    agg = jax.ops.segment_max(h_e, dst, num_segments=n)
    agg = jnp.maximum(jnp.where(jnp.isfinite(agg), agg, -jnp.inf), self_h)

    g = jnp.maximum(agg @ gW1 + gb1, 0.0)
    g = jnp.maximum(g @ gW2 + gb2, 0.0)
    h0 = g @ gW3 + gb3

    deg = jax.ops.segment_sum(jnp.ones((E,), jnp.float32), dst, num_segments=n) + 1.0
    dis = jax.lax.rsqrt(deg)[:, None]

    z1 = dis * (h0 @ cW1)
    acc1l, acc1r = _segsum(z1[:, : H // 2], z1[:, H // 2 :], src, dst, H // 2)
    h1 = jnp.maximum(dis * jnp.concatenate([acc1l, acc1r], axis=1) + cb1, 0.0)

    z2 = dis * (h1 @ cW2)
    acc2 = _segsum_es(z2, src, dst, OUT)
    h2 = jnp.maximum(dis * acc2 + cb2, 0.0)
    return h2
